# Initial kernel scaffold; baseline (speedup 1.0000x reference)
import jax
import jax.numpy as jnp
from jax.experimental import pallas as pl


def kernel(x, Wg, W1, W2):
    raise NotImplementedError("write your pallas kernel here")


# speedup vs baseline: 1.2434x; 1.2434x over previous
import functools

import jax
import jax.numpy as jnp
from jax.experimental import pallas as pl
from jax.experimental.pallas import tpu as pltpu

D_MODEL = 1024
N_EXP = 8
HID = 4096
T = 4096
TB = 512
NB = T // TB
TM = 256
PTOT = 2 * T + N_EXP * TM
NT = PTOT // TM
TN = 1024
NJ = HID // TN


def _routing_body(x_ref, wg_ref, eids_ref, ws_ref, ranks_ref, counts_ref,
                  loss_ref, cum_ref, colsum_ref):
    b = pl.program_id(0)

    @pl.when(b == 0)
    def _():
        cum_ref[...] = jnp.zeros_like(cum_ref)
        colsum_ref[...] = jnp.zeros_like(colsum_ref)

    logits = jax.lax.dot_general(
        x_ref[...], wg_ref[...], (((1,), (1,)), ((), ())),
        preferred_element_type=jnp.float32)
    lane = jax.lax.broadcasted_iota(jnp.int32, (TB, N_EXP), 1)
    m1 = jnp.max(logits, axis=1, keepdims=True)
    e1 = jnp.min(jnp.where(logits == m1, lane, N_EXP), axis=1)
    oh1 = (lane == e1[:, None]).astype(jnp.float32)
    masked = jnp.where(oh1 > 0, -jnp.inf, logits)
    m2 = jnp.max(masked, axis=1, keepdims=True)
    e2 = jnp.min(jnp.where(masked == m2, lane, N_EXP), axis=1)
    oh2 = (lane == e2[:, None]).astype(jnp.float32)

    a = jnp.exp(m2 - m1)
    w1 = 1.0 / (1.0 + a)
    w2 = a / (1.0 + a)

    p = jnp.exp(logits - m1)
    p = p / jnp.sum(p, axis=1, keepdims=True)
    colsum_ref[...] += jnp.sum(p, axis=0)[None, :]

    c = oh1 + oh2
    r = jax.lax.broadcasted_iota(jnp.int32, (TB, TB), 0)
    s = jax.lax.broadcasted_iota(jnp.int32, (TB, TB), 1)
    tri = (r > s).astype(jnp.float32)
    excl = jax.lax.dot_general(tri, c, (((1,), (0,)), ((), ())),
                               preferred_element_type=jnp.float32)
    base = excl + cum_ref[...]
    rank1 = jnp.sum(base * oh1, axis=1)
    rank2 = jnp.sum(base * oh2, axis=1)
    cum_ref[...] += jnp.sum(c, axis=0)[None, :]

    eids_ref[...] = jnp.concatenate([e1[:, None], e2[:, None]], axis=1)
    ws_ref[...] = jnp.concatenate([w1, w2], axis=1)
    ranks_ref[...] = jnp.concatenate(
        [rank1[:, None], rank2[:, None]], axis=1).astype(jnp.int32)

    @pl.when(b == NB - 1)
    def _():
        counts_ref[...] = cum_ref[...].astype(jnp.int32)
        mean = colsum_ref[...] / T
        loss_ref[...] = N_EXP * jnp.sum(mean * mean, keepdims=True)


def _routing(x_flat, Wg):
    return pl.pallas_call(
        _routing_body,
        grid=(NB,),
        in_specs=[
            pl.BlockSpec((TB, D_MODEL), lambda b: (b, 0)),
            pl.BlockSpec((N_EXP, D_MODEL), lambda b: (0, 0)),
        ],
        out_specs=[
            pl.BlockSpec((TB, 2), lambda b: (b, 0)),
            pl.BlockSpec((TB, 2), lambda b: (b, 0)),
            pl.BlockSpec((TB, 2), lambda b: (b, 0)),
            pl.BlockSpec((1, N_EXP), lambda b: (0, 0)),
            pl.BlockSpec((1, 1), lambda b: (0, 0)),
        ],
        out_shape=[
            jax.ShapeDtypeStruct((T, 2), jnp.int32),
            jax.ShapeDtypeStruct((T, 2), jnp.float32),
            jax.ShapeDtypeStruct((T, 2), jnp.int32),
            jax.ShapeDtypeStruct((1, N_EXP), jnp.int32),
            jax.ShapeDtypeStruct((1, 1), jnp.float32),
        ],
        scratch_shapes=[
            pltpu.VMEM((1, N_EXP), jnp.float32),
            pltpu.VMEM((1, N_EXP), jnp.float32),
        ],
    )(x_flat, Wg)


def _gmm_body(eid_ref, act_ref, x_ref, w1_ref, w2_ref, gw_ref, out_ref):
    i = pl.program_id(0)
    j = pl.program_id(1)

    @pl.when(j == 0)
    def _():
        out_ref[...] = jnp.zeros_like(out_ref)

    @pl.when(act_ref[i] != 0)
    def _():
        h = jax.lax.dot_general(
            x_ref[...], w1_ref[0], (((1,), (1,)), ((), ())),
            preferred_element_type=jnp.float32)
        h = h * jax.nn.sigmoid(h)
        out_ref[...] += jax.lax.dot_general(
            h, w2_ref[0], (((1,), (1,)), ((), ())),
            preferred_element_type=jnp.float32)

    @pl.when(j == NJ - 1)
    def _():
        out_ref[...] *= gw_ref[...]


def _gmm(tile_eid, tile_act, xs, W1, W2, gw):
    grid_spec = pltpu.PrefetchScalarGridSpec(
        num_scalar_prefetch=2,
        grid=(NT, NJ),
        in_specs=[
            pl.BlockSpec((TM, D_MODEL), lambda i, j, eid, act: (i, 0)),
            pl.BlockSpec((1, TN, D_MODEL), lambda i, j, eid, act: (eid[i], j, 0)),
            pl.BlockSpec((1, D_MODEL, TN), lambda i, j, eid, act: (eid[i], 0, j)),
            pl.BlockSpec((TM, 1), lambda i, j, eid, act: (i, 0)),
        ],
        out_specs=pl.BlockSpec((TM, D_MODEL), lambda i, j, eid, act: (i, 0)),
    )
    return pl.pallas_call(
        _gmm_body,
        grid_spec=grid_spec,
        out_shape=jax.ShapeDtypeStruct((PTOT, D_MODEL), jnp.float32),
    )(tile_eid, tile_act, xs, W1, W2, gw)


def kernel(x, Wg, W1, W2):
    batch, seq, d = x.shape
    x_flat = x.reshape(-1, d)

    eids, ws, ranks, counts, loss = _routing(x_flat, Wg)
    counts = counts[0]
    padded = ((counts + TM - 1) // TM) * TM
    bounds = jnp.cumsum(padded)
    offsets = bounds - padded
    dest = offsets[eids] + ranks

    flat_dest = dest.reshape(-1)
    tok_ids = jnp.broadcast_to(
        jnp.arange(T, dtype=jnp.int32)[:, None], (T, 2)).reshape(-1)
    src = jnp.zeros((PTOT,), jnp.int32).at[flat_dest].set(tok_ids)
    gw = jnp.zeros((PTOT, 1), jnp.float32).at[flat_dest, 0].set(ws.reshape(-1))

    tile_start = jnp.arange(NT, dtype=jnp.int32) * TM
    tile_eid = jnp.minimum(
        jnp.sum((tile_start[:, None] >= bounds[None, :]).astype(jnp.int32),
                axis=1), N_EXP - 1).astype(jnp.int32)
    tile_act = (tile_start < bounds[N_EXP - 1]).astype(jnp.int32)

    xs = x_flat[src]
    yw = _gmm(tile_eid, tile_act, xs, W1, W2, gw)
    out = yw[dest[:, 0]] + yw[dest[:, 1]]
    return out.reshape(batch, seq, d), loss[0, 0]
